# baseline (speedup 1.0000x reference)
import functools

import jax
import jax.numpy as jnp
from jax import lax
from jax.experimental import pallas as pl
from jax.experimental.pallas import tpu as pltpu
from jax.experimental.pallas import tpu_sc as plsc

B = 16384
F = 26
V = 100000
D = 16
ND = 38
HID = 64
ALL0 = F * D

NC = 2
NS = 16
NW = NC * NS
BF = B * F
PER_W = BF // NW
CH = 128
GRP = 8
GROUP_N = CH * GRP
N_GRP = PER_W // GROUP_N
NT = D + 1


def _sc_gather(tables, idx3):
    mesh = plsc.VectorSubcoreMesh(core_axis_name="c", subcore_axis_name="s")

    @functools.partial(
        pl.kernel,
        mesh=mesh,
        compiler_params=pltpu.CompilerParams(use_tc_tiling_on_sc=False),
        out_type=jax.ShapeDtypeStruct((NT, BF), jnp.float32),
        scratch_types=[
            pltpu.VMEM((PER_W // CH, CH), jnp.int32),
            pltpu.VMEM((NT, GROUP_N), jnp.float32),
            pltpu.SemaphoreType.DMA,
        ],
    )
    def k(*refs):
        tbl = refs[:NT]
        idx_hbm = refs[NT]
        out_hbm = refs[NT + 1]
        idx_v, buf_v, sem = refs[NT + 2], refs[NT + 3], refs[NT + 4]

        wid = lax.axis_index("s") * NC + lax.axis_index("c")
        base = wid * PER_W
        pltpu.sync_copy(idx_hbm.at[wid], idx_v)

        def group(g, _):
            def chunk_copies(c):
                ii = idx_v.at[g * GRP + c]
                return [
                    pltpu.make_async_copy(
                        tbl[t].at[ii], buf_v.at[t, pl.ds(c * CH, CH)], sem)
                    for t in range(NT)
                ]

            def fire(c, _):
                for cp in chunk_copies(c):
                    cp.start()
                return ()

            def drain(c, _):
                for cp in chunk_copies(c):
                    cp.wait()
                return ()

            lax.fori_loop(0, GRP, fire, (), unroll=False)
            lax.fori_loop(0, GRP, drain, (), unroll=False)
            off = base + g * GROUP_N
            for t in range(NT):
                pltpu.sync_copy(buf_v.at[t], out_hbm.at[t, pl.ds(off, GROUP_N)])
            return ()

        lax.fori_loop(0, N_GRP, group, (), unroll=False)

    return k(*tables, idx3)


def _dense_body(*refs):
    e2d = refs[:D]
    (e1_ref, xd_ref, smat_ref, w1d_ref, wd_ref, bd_ref,
     w1_ref, c1_ref, w2_ref, c2_ref, w3_ref, c3_ref, wo_ref,
     co_ref, o_ref) = refs[D:]
    hi = jax.lax.Precision.HIGHEST
    f32 = jnp.float32
    e2 = jnp.concatenate([r[...] for r in e2d], axis=1)
    e1 = e1_ref[...]
    xd = xd_ref[...]
    smat = smat_ref[...]
    fm1 = jnp.sum(e1, axis=1, keepdims=True) + jnp.dot(
        xd, w1d_ref[...], precision=hi, preferred_element_type=f32)
    sum_e = jnp.dot(e2, smat, precision=hi, preferred_element_type=f32)
    ssq_e = jnp.dot(e2 * e2, smat, precision=hi, preferred_element_type=f32)
    fm2 = 0.5 * jnp.sum(sum_e * sum_e - ssq_e, axis=1, keepdims=True)
    d0 = e2 + jnp.maximum(
        jnp.dot(xd, wd_ref[...], precision=hi, preferred_element_type=f32)
        + bd_ref[...], 0.0)
    h = jnp.maximum(
        jnp.dot(d0, w1_ref[...], precision=hi, preferred_element_type=f32)
        + c1_ref[...], 0.0)
    h = jnp.maximum(
        jnp.dot(h, w2_ref[...], precision=hi, preferred_element_type=f32)
        + c2_ref[...], 0.0)
    h = jnp.maximum(
        jnp.dot(h, w3_ref[...], precision=hi, preferred_element_type=f32)
        + c3_ref[...], 0.0)
    z = fm1 + fm2 + jnp.dot(h, wo_ref[...], precision=hi,
                            preferred_element_type=f32) + co_ref[...]
    o_ref[...] = jax.nn.sigmoid(z)


def _dense_tc(e2d_list, e1, xd, smat, w1dT, wdT, bd2, w1p, c1, w2p, c2,
              w3p, c3, wop, co):
    BLK = 1024
    grid = (B // BLK,)
    row = lambda i: (i, 0)
    fixed = lambda i: (0, 0)
    in_specs = (
        [pl.BlockSpec((BLK, F), row) for _ in range(D)]
        + [
            pl.BlockSpec((BLK, F), row),
            pl.BlockSpec((BLK, ND), row),
            pl.BlockSpec((ALL0, D), fixed),
            pl.BlockSpec((ND, 1), fixed),
            pl.BlockSpec((ND, ALL0), fixed),
            pl.BlockSpec((1, ALL0), fixed),
            pl.BlockSpec((ALL0, HID), fixed),
            pl.BlockSpec((1, HID), fixed),
            pl.BlockSpec((HID, HID), fixed),
            pl.BlockSpec((1, HID), fixed),
            pl.BlockSpec((HID, HID), fixed),
            pl.BlockSpec((1, HID), fixed),
            pl.BlockSpec((HID, 1), fixed),
            pl.BlockSpec((1, 1), fixed),
        ]
    )
    return pl.pallas_call(
        _dense_body,
        grid=grid,
        in_specs=in_specs,
        out_specs=pl.BlockSpec((BLK, 1), row),
        out_shape=jax.ShapeDtypeStruct((B, 1), jnp.float32),
    )(*e2d_list, e1, xd, smat, w1dT, wdT, bd2, w1p, c1, w2p, c2, w3p, c3,
      wop, co)


def kernel(X_sparse, X_dense, T1, T2, W1d, b1d, Wd, bd,
           W1, b1, g1, be1, rm1, rv1,
           W2, b2, g2, be2, rm2, rv2,
           W3, b3, g3, be3, rm3, rv3,
           Wo, bo):
    idx = (X_sparse.astype(jnp.int32)
           + (jnp.arange(F, dtype=jnp.int32) * V)[None, :])
    idx3 = idx.reshape(NW, PER_W // CH, CH)
    tables = [T2[:, :, d].reshape(F * V) for d in range(D)]
    tables.append(T1.reshape(F * V))

    gat = _sc_gather(tables, idx3)
    e2d_list = [gat[d].reshape(B, F) for d in range(D)]
    e1 = gat[D].reshape(B, F)

    def fold(Wt, bt, g, be, rm, rv):
        s = g * jax.lax.rsqrt(rv + 1e-5)
        t = be - rm * s
        return Wt.T * s[None, :], (bt * s + t)[None, :]

    w1p, c1 = fold(W1, b1, g1, be1, rm1, rv1)
    w2p, c2 = fold(W2, b2, g2, be2, rm2, rv2)
    w3p, c3 = fold(W3, b3, g3, be3, rm3, rv3)
    smat = jnp.tile(jnp.eye(D, dtype=jnp.float32), (F, 1))
    co = (b1d + bo).reshape(1, 1)

    ocol = (jnp.arange(ALL0, dtype=jnp.int32) % F) * D + (
        jnp.arange(ALL0, dtype=jnp.int32) // F)
    smat_p = jnp.take(smat, ocol, axis=0)
    wdT_p = jnp.take(Wd.T, ocol, axis=1)
    bd_p = jnp.take(bd, ocol).reshape(1, ALL0)
    w1p_p = jnp.take(w1p, ocol, axis=0)

    out = _dense_tc(e2d_list, e1, X_dense, smat_p, W1d.T, wdT_p, bd_p,
                    w1p_p, c1, w2p, c2, w3p, c3, Wo.T, co)
    return out.reshape(B)


# speedup vs baseline: 1.6750x; 1.6750x over previous
import functools

import jax
import jax.numpy as jnp
from jax import lax
from jax.experimental import pallas as pl
from jax.experimental.pallas import tpu as pltpu
from jax.experimental.pallas import tpu_sc as plsc

B = 16384
F = 26
V = 100000
VP = 100096
D = 16
ND = 38
HID = 64
ALL0 = F * D

NC = 2
NS = 16
NW = NC * NS
BF = B * F
PER_W = BF // NW
CH = 128
GRP = 4
GROUP_ROWS = CH * GRP
N_GRP = PER_W // GROUP_ROWS

VB = 5888
KB = VP // VB
RB = VB // 8


def _pack_body(x_ref, o_ref):
    y = jnp.transpose(x_ref[...])
    y8 = y.reshape(RB, 8, 16)
    cat = jnp.concatenate([y8[:, s, :] for s in range(8)], axis=1)
    o_ref[...] = cat.reshape(VB * D)


def _pack_table(t2p2d):
    return pl.pallas_call(
        _pack_body,
        grid=(F, KB),
        in_specs=[pl.BlockSpec((D, VB), lambda f, k: (f, k))],
        out_specs=pl.BlockSpec((VB * D,), lambda f, k: (f * KB + k,)),
        out_shape=jax.ShapeDtypeStruct((F * VP * D,), jnp.float32),
    )(t2p2d)


def _sc_gather(t2_flat, t1_flat, idx2, idx1):
    mesh = plsc.VectorSubcoreMesh(core_axis_name="c", subcore_axis_name="s")

    @functools.partial(
        pl.kernel,
        mesh=mesh,
        compiler_params=pltpu.CompilerParams(use_tc_tiling_on_sc=False),
        out_type=(
            jax.ShapeDtypeStruct((BF, D), jnp.float32),
            jax.ShapeDtypeStruct((BF,), jnp.float32),
        ),
        scratch_types=[
            pltpu.VMEM((PER_W // CH, CH), jnp.int32),
            pltpu.VMEM((PER_W // CH, CH), jnp.int32),
            pltpu.VMEM((GROUP_ROWS, D), jnp.float32),
            pltpu.VMEM((GROUP_ROWS,), jnp.float32),
            pltpu.SemaphoreType.DMA,
        ],
    )
    def k(t2_hbm, t1_hbm, idx2_hbm, idx1_hbm, out2_hbm, out1_hbm,
          idx2_v, idx1_v, rows_v, scal_v, sem):
        wid = lax.axis_index("s") * NC + lax.axis_index("c")
        base = wid * PER_W
        pltpu.sync_copy(idx2_hbm.at[wid], idx2_v)
        pltpu.sync_copy(idx1_hbm.at[wid], idx1_v)

        def body(g, _):
            cps = []
            for j in range(GRP):
                cps.append(pltpu.make_async_copy(
                    t2_hbm.at[idx2_v.at[g * GRP + j]],
                    rows_v.at[pl.ds(j * CH, CH)], sem))
                cps.append(pltpu.make_async_copy(
                    t1_hbm.at[idx1_v.at[g * GRP + j]],
                    scal_v.at[pl.ds(j * CH, CH)], sem))
            for c in cps:
                c.start()
            for c in cps:
                c.wait()
            off = base + g * GROUP_ROWS
            pltpu.sync_copy(rows_v, out2_hbm.at[pl.ds(off, GROUP_ROWS)])
            pltpu.sync_copy(scal_v, out1_hbm.at[pl.ds(off, GROUP_ROWS)])
            return ()

        lax.fori_loop(0, N_GRP, body, (), unroll=False)

    return k(t2_flat, t1_flat, idx2, idx1)


def _dense_body(e2_ref, e1_ref, xd_ref, smat_ref, w1d_ref, wd_ref, bd_ref,
                w1_ref, c1_ref, w2_ref, c2_ref, w3_ref, c3_ref, wo_ref,
                co_ref, o_ref):
    hi = jax.lax.Precision.HIGHEST
    f32 = jnp.float32
    e2 = e2_ref[...]
    e1 = e1_ref[...]
    xd = xd_ref[...]
    smat = smat_ref[...]
    fm1 = jnp.sum(e1, axis=1, keepdims=True) + jnp.dot(
        xd, w1d_ref[...], precision=hi, preferred_element_type=f32)
    sum_e = jnp.dot(e2, smat, precision=hi, preferred_element_type=f32)
    ssq_e = jnp.dot(e2 * e2, smat, precision=hi, preferred_element_type=f32)
    fm2 = 0.5 * jnp.sum(sum_e * sum_e - ssq_e, axis=1, keepdims=True)
    d0 = e2 + jnp.maximum(
        jnp.dot(xd, wd_ref[...], precision=hi, preferred_element_type=f32)
        + bd_ref[...], 0.0)
    h = jnp.maximum(
        jnp.dot(d0, w1_ref[...], precision=hi, preferred_element_type=f32)
        + c1_ref[...], 0.0)
    h = jnp.maximum(
        jnp.dot(h, w2_ref[...], precision=hi, preferred_element_type=f32)
        + c2_ref[...], 0.0)
    h = jnp.maximum(
        jnp.dot(h, w3_ref[...], precision=hi, preferred_element_type=f32)
        + c3_ref[...], 0.0)
    z = fm1 + fm2 + jnp.dot(h, wo_ref[...], precision=hi,
                            preferred_element_type=f32) + co_ref[...]
    o_ref[...] = jax.nn.sigmoid(z)


def _dense_tc(e2, e1, xd, smat, w1dT, wdT, bd2, w1p, c1, w2p, c2, w3p, c3,
              wop, co):
    BLK = 1024
    grid = (B // BLK,)
    row = lambda i: (i, 0)
    fixed = lambda i: (0, 0)
    in_specs = [
        pl.BlockSpec((BLK, ALL0), row),
        pl.BlockSpec((BLK, F), row),
        pl.BlockSpec((BLK, ND), row),
        pl.BlockSpec((ALL0, D), fixed),
        pl.BlockSpec((ND, 1), fixed),
        pl.BlockSpec((ND, ALL0), fixed),
        pl.BlockSpec((1, ALL0), fixed),
        pl.BlockSpec((ALL0, HID), fixed),
        pl.BlockSpec((1, HID), fixed),
        pl.BlockSpec((HID, HID), fixed),
        pl.BlockSpec((1, HID), fixed),
        pl.BlockSpec((HID, HID), fixed),
        pl.BlockSpec((1, HID), fixed),
        pl.BlockSpec((HID, 1), fixed),
        pl.BlockSpec((1, 1), fixed),
    ]
    return pl.pallas_call(
        _dense_body,
        grid=grid,
        in_specs=in_specs,
        out_specs=pl.BlockSpec((BLK, 1), row),
        out_shape=jax.ShapeDtypeStruct((B, 1), jnp.float32),
    )(e2, e1, xd, smat, w1dT, wdT, bd2, w1p, c1, w2p, c2, w3p, c3, wop, co)


def kernel(X_sparse, X_dense, T1, T2, W1d, b1d, Wd, bd,
           W1, b1, g1, be1, rm1, rv1,
           W2, b2, g2, be2, rm2, rv2,
           W3, b3, g3, be3, rm3, rv3,
           Wo, bo):
    xs = X_sparse.astype(jnp.int32)
    fr = jnp.arange(F, dtype=jnp.int32)[None, :]
    idx2 = (xs + fr * VP).reshape(NW, PER_W // CH, CH)
    idx1 = (xs + fr * V).reshape(NW, PER_W // CH, CH)

    t2p2d = jnp.transpose(T2, (0, 2, 1)).reshape(F * D, V)
    t2_flat = _pack_table(t2p2d).reshape(F * VP, D)
    t1_flat = T1.reshape(F * V)

    rows, scal = _sc_gather(t2_flat, t1_flat, idx2, idx1)
    e2 = rows.reshape(B, ALL0)
    e1 = scal.reshape(B, F)

    def fold(Wt, bt, g, be, rm, rv):
        s = g * jax.lax.rsqrt(rv + 1e-5)
        t = be - rm * s
        return Wt.T * s[None, :], (bt * s + t)[None, :]

    w1p, c1 = fold(W1, b1, g1, be1, rm1, rv1)
    w2p, c2 = fold(W2, b2, g2, be2, rm2, rv2)
    w3p, c3 = fold(W3, b3, g3, be3, rm3, rv3)
    smat = jnp.tile(jnp.eye(D, dtype=jnp.float32), (F, 1))
    co = (b1d + bo).reshape(1, 1)

    out = _dense_tc(e2, e1, X_dense, smat, W1d.T, Wd.T, bd.reshape(1, ALL0),
                    w1p, c1, w2p, c2, w3p, c3, Wo.T, co)
    return out.reshape(B)


# speedup vs baseline: 1.9075x; 1.1388x over previous
import functools

import jax
import jax.numpy as jnp
from jax import lax
from jax.experimental import pallas as pl
from jax.experimental.pallas import tpu as pltpu
from jax.experimental.pallas import tpu_sc as plsc

B = 16384
F = 26
V = 100000
VP = 100096
D = 16
ND = 38
HID = 64
ALL0 = F * D

NC = 2
NS = 16
NW = NC * NS
BF = B * F
PER_W = BF // NW
CH = 128
GRP = 4
GROUP_ROWS = CH * GRP
N_GRP = PER_W // GROUP_ROWS

VB = 5888
KB = VP // VB
RB = VB // 8


def _pack_body(x_ref, o_ref):
    y = jnp.transpose(x_ref[...])
    y8 = y.reshape(RB, 8, 16)
    cat = jnp.concatenate([y8[:, s, :] for s in range(8)], axis=1)
    o_ref[...] = cat.reshape(VB * D)


def _pack_table(t2p2d):
    return pl.pallas_call(
        _pack_body,
        grid=(F, KB),
        in_specs=[pl.BlockSpec((D, VB), lambda f, k: (f, k))],
        out_specs=pl.BlockSpec((VB * D,), lambda f, k: (f * KB + k,)),
        out_shape=jax.ShapeDtypeStruct((F * VP * D,), jnp.float32),
    )(t2p2d)


def _sc_gather(t2_flat, t1_flat, idx2, idx1):
    mesh = plsc.VectorSubcoreMesh(core_axis_name="c", subcore_axis_name="s")

    @functools.partial(
        pl.kernel,
        mesh=mesh,
        compiler_params=pltpu.CompilerParams(use_tc_tiling_on_sc=False),
        out_type=(
            jax.ShapeDtypeStruct((BF, D), jnp.float32),
            jax.ShapeDtypeStruct((BF,), jnp.float32),
        ),
        scratch_types=[
            pltpu.VMEM((PER_W // CH, CH), jnp.int32),
            pltpu.VMEM((PER_W // CH, CH), jnp.int32),
            pltpu.VMEM((GROUP_ROWS, D), jnp.float32),
            pltpu.VMEM((GROUP_ROWS,), jnp.float32),
            pltpu.SemaphoreType.DMA,
        ],
    )
    def k(t2_hbm, t1_hbm, idx2_hbm, idx1_hbm, out2_hbm, out1_hbm,
          idx2_v, idx1_v, rows_v, scal_v, sem):
        wid = lax.axis_index("s") * NC + lax.axis_index("c")
        base = wid * PER_W
        pltpu.sync_copy(idx2_hbm.at[wid], idx2_v)
        pltpu.sync_copy(idx1_hbm.at[wid], idx1_v)

        def body(g, _):
            cps = []
            for j in range(GRP):
                cps.append(pltpu.make_async_copy(
                    t2_hbm.at[idx2_v.at[g * GRP + j]],
                    rows_v.at[pl.ds(j * CH, CH)], sem))
                cps.append(pltpu.make_async_copy(
                    t1_hbm.at[idx1_v.at[g * GRP + j]],
                    scal_v.at[pl.ds(j * CH, CH)], sem))
            for c in cps:
                c.start()
            for c in cps:
                c.wait()
            off = base + g * GROUP_ROWS
            pltpu.sync_copy(rows_v, out2_hbm.at[pl.ds(off, GROUP_ROWS)])
            pltpu.sync_copy(scal_v, out1_hbm.at[pl.ds(off, GROUP_ROWS)])
            return ()

        lax.fori_loop(0, N_GRP, body, (), unroll=False)

    return k(t2_flat, t1_flat, idx2, idx1)


def _dense_body(e2_ref, e1_ref, xd_ref, smat_ref, w1d_ref, wd_ref, bd_ref,
                w1_ref, c1_ref, w2_ref, c2_ref, w3_ref, c3_ref, wo_ref,
                co_ref, o_ref):
    hi = jax.lax.Precision.DEFAULT
    f32 = jnp.float32
    e2 = e2_ref[...]
    e1 = e1_ref[...]
    xd = xd_ref[...]
    smat = smat_ref[...]
    fm1 = jnp.sum(e1, axis=1, keepdims=True) + jnp.dot(
        xd, w1d_ref[...], precision=hi, preferred_element_type=f32)
    sum_e = jnp.dot(e2, smat, precision=hi, preferred_element_type=f32)
    ssq_e = jnp.dot(e2 * e2, smat, precision=hi, preferred_element_type=f32)
    fm2 = 0.5 * jnp.sum(sum_e * sum_e - ssq_e, axis=1, keepdims=True)
    d0 = e2 + jnp.maximum(
        jnp.dot(xd, wd_ref[...], precision=hi, preferred_element_type=f32)
        + bd_ref[...], 0.0)
    h = jnp.maximum(
        jnp.dot(d0, w1_ref[...], precision=hi, preferred_element_type=f32)
        + c1_ref[...], 0.0)
    h = jnp.maximum(
        jnp.dot(h, w2_ref[...], precision=hi, preferred_element_type=f32)
        + c2_ref[...], 0.0)
    h = jnp.maximum(
        jnp.dot(h, w3_ref[...], precision=hi, preferred_element_type=f32)
        + c3_ref[...], 0.0)
    z = fm1 + fm2 + jnp.dot(h, wo_ref[...], precision=hi,
                            preferred_element_type=f32) + co_ref[...]
    o_ref[...] = jax.nn.sigmoid(z)


def _dense_tc(e2, e1, xd, smat, w1dT, wdT, bd2, w1p, c1, w2p, c2, w3p, c3,
              wop, co):
    BLK = 1024
    grid = (B // BLK,)
    row = lambda i: (i, 0)
    fixed = lambda i: (0, 0)
    in_specs = [
        pl.BlockSpec((BLK, ALL0), row),
        pl.BlockSpec((BLK, F), row),
        pl.BlockSpec((BLK, ND), row),
        pl.BlockSpec((ALL0, D), fixed),
        pl.BlockSpec((ND, 1), fixed),
        pl.BlockSpec((ND, ALL0), fixed),
        pl.BlockSpec((1, ALL0), fixed),
        pl.BlockSpec((ALL0, HID), fixed),
        pl.BlockSpec((1, HID), fixed),
        pl.BlockSpec((HID, HID), fixed),
        pl.BlockSpec((1, HID), fixed),
        pl.BlockSpec((HID, HID), fixed),
        pl.BlockSpec((1, HID), fixed),
        pl.BlockSpec((HID, 1), fixed),
        pl.BlockSpec((1, 1), fixed),
    ]
    return pl.pallas_call(
        _dense_body,
        grid=grid,
        in_specs=in_specs,
        out_specs=pl.BlockSpec((BLK, 1), row),
        out_shape=jax.ShapeDtypeStruct((B, 1), jnp.float32),
    )(e2, e1, xd, smat, w1dT, wdT, bd2, w1p, c1, w2p, c2, w3p, c3, wop, co)


def kernel(X_sparse, X_dense, T1, T2, W1d, b1d, Wd, bd,
           W1, b1, g1, be1, rm1, rv1,
           W2, b2, g2, be2, rm2, rv2,
           W3, b3, g3, be3, rm3, rv3,
           Wo, bo):
    xs = X_sparse.astype(jnp.int32)
    fr = jnp.arange(F, dtype=jnp.int32)[None, :]
    idx2 = (xs + fr * VP).reshape(NW, PER_W // CH, CH)
    idx1 = (xs + fr * V).reshape(NW, PER_W // CH, CH)

    t2p2d = jnp.transpose(T2, (0, 2, 1)).reshape(F * D, V)
    t2_flat = _pack_table(t2p2d).reshape(F * VP, D)
    t1_flat = T1.reshape(F * V)

    rows, scal = _sc_gather(t2_flat, t1_flat, idx2, idx1)
    e2 = rows.reshape(B, ALL0)
    e1 = scal.reshape(B, F)

    def fold(Wt, bt, g, be, rm, rv):
        s = g * jax.lax.rsqrt(rv + 1e-5)
        t = be - rm * s
        return Wt.T * s[None, :], (bt * s + t)[None, :]

    w1p, c1 = fold(W1, b1, g1, be1, rm1, rv1)
    w2p, c2 = fold(W2, b2, g2, be2, rm2, rv2)
    w3p, c3 = fold(W3, b3, g3, be3, rm3, rv3)
    smat = jnp.tile(jnp.eye(D, dtype=jnp.float32), (F, 1))
    co = (b1d + bo).reshape(1, 1)

    out = _dense_tc(e2, e1, X_dense, smat, W1d.T, Wd.T, bd.reshape(1, ALL0),
                    w1p, c1, w2p, c2, w3p, c3, Wo.T, co)
    return out.reshape(B)


# speedup vs baseline: 1.9111x; 1.0019x over previous
import functools

import jax
import jax.numpy as jnp
from jax import lax
from jax.experimental import pallas as pl
from jax.experimental.pallas import tpu as pltpu
from jax.experimental.pallas import tpu_sc as plsc

B = 16384
F = 26
V = 100000
VP = 100096
D = 16
ND = 38
HID = 64
ALL0 = F * D

NC = 2
NS = 16
NW = NC * NS
BF = B * F
PER_W = BF // NW
CH = 128
GRP = 4
GROUP_ROWS = CH * GRP
N_GRP = PER_W // GROUP_ROWS

VB = 5888
KB = VP // VB
RB = VB // 8


def _pack_body(x_ref, o_ref):
    y = jnp.transpose(x_ref[...])
    y8 = y.reshape(RB, 8, 16)
    cat = jnp.concatenate([y8[:, s, :] for s in range(8)], axis=1)
    o_ref[...] = cat.reshape(VB * D)


def _pack_table(t2p2d):
    return pl.pallas_call(
        _pack_body,
        grid=(F, KB),
        in_specs=[pl.BlockSpec((D, VB), lambda f, k: (f, k))],
        out_specs=pl.BlockSpec((VB * D,), lambda f, k: (f * KB + k,)),
        out_shape=jax.ShapeDtypeStruct((F * VP * D,), jnp.float32),
    )(t2p2d)


def _sc_gather(t2_flat, t1_flat, idx2, idx1):
    mesh = plsc.VectorSubcoreMesh(core_axis_name="c", subcore_axis_name="s")

    @functools.partial(
        pl.kernel,
        mesh=mesh,
        compiler_params=pltpu.CompilerParams(use_tc_tiling_on_sc=False),
        out_type=(
            jax.ShapeDtypeStruct((BF, D), jnp.float32),
            jax.ShapeDtypeStruct((BF,), jnp.float32),
        ),
        scratch_types=[
            pltpu.VMEM((PER_W // CH, CH), jnp.int32),
            pltpu.VMEM((PER_W // CH, CH), jnp.int32),
            pltpu.VMEM((GROUP_ROWS, D), jnp.float32),
            pltpu.VMEM((GROUP_ROWS,), jnp.float32),
            pltpu.SemaphoreType.DMA,
        ],
    )
    def k(t2_hbm, t1_hbm, idx2_hbm, idx1_hbm, out2_hbm, out1_hbm,
          idx2_v, idx1_v, rows_v, scal_v, sem):
        wid = lax.axis_index("s") * NC + lax.axis_index("c")
        base = wid * PER_W
        pltpu.sync_copy(idx2_hbm.at[wid], idx2_v)
        pltpu.sync_copy(idx1_hbm.at[wid], idx1_v)

        def body(g, _):
            cps = []
            for j in range(GRP):
                cps.append(pltpu.make_async_copy(
                    t2_hbm.at[idx2_v.at[g * GRP + j]],
                    rows_v.at[pl.ds(j * CH, CH)], sem))
                cps.append(pltpu.make_async_copy(
                    t1_hbm.at[idx1_v.at[g * GRP + j]],
                    scal_v.at[pl.ds(j * CH, CH)], sem))
            for c in cps:
                c.start()
            for c in cps:
                c.wait()
            off = base + g * GROUP_ROWS
            pltpu.sync_copy(rows_v, out2_hbm.at[pl.ds(off, GROUP_ROWS)])
            pltpu.sync_copy(scal_v, out1_hbm.at[pl.ds(off, GROUP_ROWS)])
            return ()

        lax.fori_loop(0, N_GRP, body, (), unroll=False)

    return k(t2_flat, t1_flat, idx2, idx1)


def _dense_body(e2_ref, e1_ref, xd_ref, smat_ref, w1d_ref, wd_ref, bd_ref,
                w1_ref, c1_ref, w2_ref, c2_ref, w3_ref, c3_ref, wo_ref,
                co_ref, o_ref):
    hi = jax.lax.Precision.DEFAULT
    f32 = jnp.float32
    e2 = e2_ref[...]
    e1 = e1_ref[...]
    xd = xd_ref[...]
    smat = smat_ref[...]
    fm1 = jnp.sum(e1, axis=1, keepdims=True) + jnp.dot(
        xd, w1d_ref[...], precision=hi, preferred_element_type=f32)
    sum_e = jnp.dot(e2, smat, precision=hi, preferred_element_type=f32)
    ssq_e = jnp.dot(e2 * e2, smat, precision=hi, preferred_element_type=f32)
    fm2 = 0.5 * jnp.sum(sum_e * sum_e - ssq_e, axis=1, keepdims=True)
    d0 = e2 + jnp.maximum(
        jnp.dot(xd, wd_ref[...], precision=hi, preferred_element_type=f32)
        + bd_ref[...], 0.0)
    h = jnp.maximum(
        jnp.dot(d0, w1_ref[...], precision=hi, preferred_element_type=f32)
        + c1_ref[...], 0.0)
    h = jnp.maximum(
        jnp.dot(h, w2_ref[...], precision=hi, preferred_element_type=f32)
        + c2_ref[...], 0.0)
    h = jnp.maximum(
        jnp.dot(h, w3_ref[...], precision=hi, preferred_element_type=f32)
        + c3_ref[...], 0.0)
    z = fm1 + fm2 + jnp.dot(h, wo_ref[...], precision=hi,
                            preferred_element_type=f32) + co_ref[...]
    o_ref[...] = jax.nn.sigmoid(z)


def _dense_tc(e2, e1, xd, smat, w1dT, wdT, bd2, w1p, c1, w2p, c2, w3p, c3,
              wop, co):
    BLK = 2048
    grid = (B // BLK,)
    row = lambda i: (i, 0)
    fixed = lambda i: (0, 0)
    in_specs = [
        pl.BlockSpec((BLK, ALL0), row),
        pl.BlockSpec((BLK, F), row),
        pl.BlockSpec((BLK, ND), row),
        pl.BlockSpec((ALL0, D), fixed),
        pl.BlockSpec((ND, 1), fixed),
        pl.BlockSpec((ND, ALL0), fixed),
        pl.BlockSpec((1, ALL0), fixed),
        pl.BlockSpec((ALL0, HID), fixed),
        pl.BlockSpec((1, HID), fixed),
        pl.BlockSpec((HID, HID), fixed),
        pl.BlockSpec((1, HID), fixed),
        pl.BlockSpec((HID, HID), fixed),
        pl.BlockSpec((1, HID), fixed),
        pl.BlockSpec((HID, 1), fixed),
        pl.BlockSpec((1, 1), fixed),
    ]
    return pl.pallas_call(
        _dense_body,
        grid=grid,
        in_specs=in_specs,
        out_specs=pl.BlockSpec((BLK, 1), row),
        out_shape=jax.ShapeDtypeStruct((B, 1), jnp.float32),
    )(e2, e1, xd, smat, w1dT, wdT, bd2, w1p, c1, w2p, c2, w3p, c3, wop, co)


def kernel(X_sparse, X_dense, T1, T2, W1d, b1d, Wd, bd,
           W1, b1, g1, be1, rm1, rv1,
           W2, b2, g2, be2, rm2, rv2,
           W3, b3, g3, be3, rm3, rv3,
           Wo, bo):
    xs = X_sparse.astype(jnp.int32)
    fr = jnp.arange(F, dtype=jnp.int32)[None, :]
    idx2 = (xs + fr * VP).reshape(NW, PER_W // CH, CH)
    idx1 = (xs + fr * V).reshape(NW, PER_W // CH, CH)

    t2p2d = jnp.transpose(T2, (0, 2, 1)).reshape(F * D, V)
    t2_flat = _pack_table(t2p2d).reshape(F * VP, D)
    t1_flat = T1.reshape(F * V)

    rows, scal = _sc_gather(t2_flat, t1_flat, idx2, idx1)
    e2 = rows.reshape(B, ALL0)
    e1 = scal.reshape(B, F)

    def fold(Wt, bt, g, be, rm, rv):
        s = g * jax.lax.rsqrt(rv + 1e-5)
        t = be - rm * s
        return Wt.T * s[None, :], (bt * s + t)[None, :]

    w1p, c1 = fold(W1, b1, g1, be1, rm1, rv1)
    w2p, c2 = fold(W2, b2, g2, be2, rm2, rv2)
    w3p, c3 = fold(W3, b3, g3, be3, rm3, rv3)
    smat = jnp.tile(jnp.eye(D, dtype=jnp.float32), (F, 1))
    co = (b1d + bo).reshape(1, 1)

    out = _dense_tc(e2, e1, X_dense, smat, W1d.T, Wd.T, bd.reshape(1, ALL0),
                    w1p, c1, w2p, c2, w3p, c3, Wo.T, co)
    return out.reshape(B)
